# baseline (speedup 1.0000x reference)
import functools

import jax
import jax.numpy as jnp
from jax import lax
from jax.experimental import pallas as pl
from jax.experimental.pallas import tpu as pltpu
from jax.experimental.pallas import tpu_sc as plsc

D = 64
SEQ = 200
LANES = 16
NBUF = 4


@jax.jit
def kernel(inputs, token_table, pos_table):
    B, S = inputs.shape
    assert S == SEQ and token_table.shape[1] == D

    info = plsc.get_sparse_core_info()
    nw = info.num_cores * info.num_subcores
    b_per_w = B // nw

    mesh = plsc.VectorSubcoreMesh(core_axis_name="c", subcore_axis_name="s")

    @functools.partial(
        pl.kernel,
        mesh=mesh,
        out_type=jax.ShapeDtypeStruct((B, S, D), jnp.float32),
        compiler_params=pltpu.CompilerParams(use_tc_tiling_on_sc=False),
        scratch_types=[
            pltpu.VMEM((NBUF * SEQ,), jnp.int32),
            pltpu.VMEM((SEQ, D), jnp.float32),
            pltpu.VMEM((NBUF, SEQ, D), jnp.float32),
        ] + [pltpu.SemaphoreType.DMA] * (3 * NBUF),
    )
    def sc_embed(idx_hbm, table_hbm, pos_hbm, out_hbm,
                 idx_v, pos_v, rows_v, *sems):
        isem = sems[:NBUF]
        gsem = sems[NBUF:2 * NBUF]
        ssem = sems[2 * NBUF:]
        wid = lax.axis_index("s") * info.num_cores + lax.axis_index("c")
        base = wid * b_per_w
        pltpu.sync_copy(pos_hbm, pos_v)

        def idx_copy(g, slot):
            return pltpu.make_async_copy(
                idx_hbm.at[base + g],
                idx_v.at[pl.ds(slot * SEQ, SEQ)], isem[slot])

        def gather_copies(slot):
            buf = rows_v.at[slot]
            off = slot * SEQ
            return (
                pltpu.make_async_copy(
                    table_hbm.at[idx_v.at[pl.ds(off, 104)]],
                    buf.at[pl.ds(0, 104)], gsem[slot]),
                pltpu.make_async_copy(
                    table_hbm.at[idx_v.at[pl.ds(off + 104, 96)]],
                    buf.at[pl.ds(104, 96)], gsem[slot]),
            )

        def store_copy(g, slot):
            return pltpu.make_async_copy(
                rows_v.at[slot], out_hbm.at[base + g], ssem[slot])

        def start_gather(slot):
            for cp in gather_copies(slot):
                cp.start()

        for g0 in range(3):
            idx_copy(g0, g0).start()
        for g0 in range(2):
            idx_copy(g0, g0).wait()
            start_gather(g0)

        def quad(i, carry):
            for k in range(NBUF):
                g = i * NBUF + k
                buf = rows_v.at[k]
                for cp in gather_copies(k):
                    cp.wait()
                k3 = (k + 3) % NBUF

                @pl.when(g + 3 < b_per_w)
                def _():
                    idx_copy(g + 3, k3).start()

                k2 = (k + 2) % NBUF

                @pl.when(g + 2 < b_per_w)
                def _():
                    idx_copy(0, k2).wait()

                    @pl.when(g >= 2)
                    def _():
                        store_copy(0, k2).wait()
                    start_gather(k2)

                def row(r, c):
                    for j in range(D // LANES):
                        sl = pl.ds(j * LANES, LANES)
                        plsc.addupdate(buf.at[r, sl], pos_v[r, sl])
                    return c

                lax.fori_loop(0, SEQ, row, 0, unroll=2)
                store_copy(g, k).start()
            return carry

        lax.fori_loop(0, b_per_w // NBUF, quad, 0)
        for k in range(NBUF):
            store_copy(0, k).wait()

    return sc_embed(inputs, token_table, pos_table)
